# baseline (speedup 1.0000x reference)
import jax
import jax.numpy as jnp
from jax.experimental import pallas as pl

TOPK = 8
BT = 1024
CHUNK = 512


def _gate_kernel(x_ref, wt_ref, w_out_ref, i_out_ref):
    wt = wt_ref[...]
    e = wt.shape[1]

    def body(c):
        r = pl.ds(c * CHUNK, CHUNK)
        x = x_ref[r, :]
        scores = jnp.dot(x, wt, preferred_element_type=jnp.float32)
        m = jnp.max(scores, axis=-1, keepdims=True)
        z = jnp.sum(jnp.exp(scores - m), axis=-1, keepdims=True)
        iota_f = jax.lax.broadcasted_iota(jnp.int32, scores.shape, 1).astype(
            jnp.float32)
        s = scores
        vals, idxs = [], []
        for _ in range(TOPK):
            cur = jnp.max(s, axis=-1, keepdims=True)
            hit = s == cur
            idxf = jnp.min(jnp.where(hit, iota_f, float(e)), axis=-1,
                           keepdims=True)
            vals.append(cur)
            idxs.append(idxf)
            s = jnp.where(iota_f == idxf, -jnp.inf, s)
        v = jnp.concatenate(vals, axis=1)
        ii = jnp.concatenate(idxs, axis=1).astype(jnp.int32)
        w_out_ref[r, :] = jnp.exp(v - m) / z
        i_out_ref[r, :] = ii

    for c in range(BT // CHUNK):
        body(c)


def kernel(x, weight):
    t, d = x.shape
    e = weight.shape[0]
    wt = weight.T
    w_out, i_out = pl.pallas_call(
        _gate_kernel,
        grid=(t // BT,),
        in_specs=[
            pl.BlockSpec((BT, d), lambda i: (i, 0)),
            pl.BlockSpec((d, e), lambda i: (0, 0)),
        ],
        out_specs=[
            pl.BlockSpec((BT, TOPK), lambda i: (i, 0)),
            pl.BlockSpec((BT, TOPK), lambda i: (i, 0)),
        ],
        out_shape=[
            jax.ShapeDtypeStruct((t, TOPK), jnp.float32),
            jax.ShapeDtypeStruct((t, TOPK), jnp.int32),
        ],
    )(x, wt)
    return w_out, i_out


# speedup vs baseline: 1.0415x; 1.0415x over previous
import jax
import jax.numpy as jnp
from jax.experimental import pallas as pl

TOPK = 8
BT = 1024
CHUNK = 128


def _gate_kernel(x_ref, wt_ref, w_out_ref, i_out_ref):
    wt = wt_ref[...]
    e = wt.shape[1]

    def body(c):
        r = pl.ds(c * CHUNK, CHUNK)
        x = x_ref[r, :]
        scores = jnp.dot(x, wt, preferred_element_type=jnp.float32)
        m = jnp.max(scores, axis=-1, keepdims=True)
        z = jnp.sum(jnp.exp(scores - m), axis=-1, keepdims=True)
        iota_f = jax.lax.broadcasted_iota(jnp.int32, scores.shape, 1).astype(
            jnp.float32)
        s = scores
        vals, idxs = [], []
        for _ in range(TOPK):
            cur = jnp.max(s, axis=-1, keepdims=True)
            hit = s == cur
            idxf = jnp.min(jnp.where(hit, iota_f, float(e)), axis=-1,
                           keepdims=True)
            vals.append(cur)
            idxs.append(idxf)
            s = jnp.where(iota_f == idxf, -jnp.inf, s)
        v = jnp.concatenate(vals, axis=1)
        ii = jnp.concatenate(idxs, axis=1).astype(jnp.int32)
        w_out_ref[r, :] = jnp.exp(v - m) / z
        i_out_ref[r, :] = ii

    for c in range(BT // CHUNK):
        body(c)


def kernel(x, weight):
    t, d = x.shape
    e = weight.shape[0]
    wt = weight.T
    w_out, i_out = pl.pallas_call(
        _gate_kernel,
        grid=(t // BT,),
        in_specs=[
            pl.BlockSpec((BT, d), lambda i: (i, 0)),
            pl.BlockSpec((d, e), lambda i: (0, 0)),
        ],
        out_specs=[
            pl.BlockSpec((BT, TOPK), lambda i: (i, 0)),
            pl.BlockSpec((BT, TOPK), lambda i: (i, 0)),
        ],
        out_shape=[
            jax.ShapeDtypeStruct((t, TOPK), jnp.float32),
            jax.ShapeDtypeStruct((t, TOPK), jnp.int32),
        ],
    )(x, wt)
    return w_out, i_out
